# Initial kernel scaffold; baseline (speedup 1.0000x reference)
import jax
import jax.numpy as jnp
from jax.experimental import pallas as pl


def kernel(r, W1, asrc1, adst1, b1, g1, o1, W2, asrc2, adst2, b2, g2, o2, W3, asrc3, adst3, b3, g3, o3, W4, asrc4, adst4, b4):
    raise NotImplementedError("write your pallas kernel here")


# speedup vs baseline: 20.1428x; 20.1428x over previous
import functools

import jax
import jax.numpy as jnp
from jax import lax
from jax.experimental import pallas as pl
from jax.experimental.pallas import tpu as pltpu
from jax.experimental.pallas import tpu_sc as plsc

N_NODE = 5000
N_FEAT = 128
ED = 32
MAXN = 16
N_EDGE = MAXN * (N_NODE - 1)
SLEFT = N_NODE * N_FEAT
B = 4
N = B * N_NODE
E = B * N_EDGE

G = 128
NROWS = 2560
E_PAD = NROWS * G
PAD = E_PAD - E
N_ACC = 20224
RPT = N_ACC // 16
ZR = 158
GROUPS_PER_TILE = NROWS // 32


def _sc_body(C, es_hbm, ed_hbm, snd_hbm, rcv_hbm, h_hbm, acc_out, den_out,
             snd_v, rcv_v, ex_v, esg_v, edg_v, rows_v, zbuf, dz,
             es_sh, ed_sh, acc_sh, den_sh, sem, sem2):
    cidx = lax.axis_index("c")
    sidx = lax.axis_index("s")
    row0 = sidx * RPT

    zvec = jnp.zeros((16,), jnp.float32)

    def _zb(i, carry):
        for c in range(C // 16):
            zbuf[i, pl.ds(c * 16, 16)] = zvec
        return carry
    lax.fori_loop(0, ZR, _zb, None)

    def _zd(i, carry):
        dz[pl.ds(i * 16, 16)] = zvec
        return carry
    lax.fori_loop(0, RPT // 16, _zd, None)

    for t in range(RPT // ZR):
        pltpu.sync_copy(zbuf, acc_sh.at[pl.ds(row0 + t * ZR, ZR)])
    pltpu.sync_copy(dz, den_sh.at[pl.ds(row0, RPT)])
    @pl.when(sidx == 0)
    def _stage():
        pltpu.sync_copy(es_hbm, es_sh.at[pl.ds(0, N)])
        pltpu.sync_copy(ed_hbm, ed_sh.at[pl.ds(0, N)])
        pltpu.sync_copy(dz.at[pl.ds(0, N_ACC - N)], es_sh.at[pl.ds(N, N_ACC - N)])
        pltpu.sync_copy(dz.at[pl.ds(0, N_ACC - N)], ed_sh.at[pl.ds(N, N_ACC - N)])
    plsc.subcore_barrier()

    base = cidx * (NROWS // 2) + sidx * GROUPS_PER_TILE

    def _grp(g, carry):
        row = base + g
        pltpu.sync_copy(snd_hbm.at[row], snd_v)
        pltpu.sync_copy(rcv_hbm.at[row], rcv_v)
        cp = pltpu.async_copy(h_hbm.at[snd_v], rows_v, sem)
        ces = pltpu.async_copy(es_sh.at[snd_v], esg_v, sem2)
        ced = pltpu.async_copy(ed_sh.at[rcv_v], edg_v, sem2)
        ces.wait()
        ced.wait()
        for j in range(G // 16):
            e = esg_v[pl.ds(j * 16, 16)] + edg_v[pl.ds(j * 16, 16)]
            e = jnp.maximum(e, 0.2 * e)
            ex_v[pl.ds(j * 16, 16)] = jnp.exp(e)
        pltpu.sync_copy(ex_v, den_sh.at[rcv_v], add=True)
        cp.wait()

        def _scale(j, carry2):
            exv = ex_v[pl.ds(j * 16, 16)]
            for kk in range(16):
                exb = jnp.broadcast_to(exv[kk], (16,))
                k = j * 16 + kk
                for c in range(C // 16):
                    rows_v[k, pl.ds(c * 16, 16)] = (
                        rows_v[k, pl.ds(c * 16, 16)] * exb)
            return carry2
        lax.fori_loop(0, G // 16, _scale, None)
        pltpu.sync_copy(rows_v, acc_sh.at[rcv_v], add=True)
        return carry

    lax.fori_loop(0, GROUPS_PER_TILE, _grp, None)
    plsc.subcore_barrier()

    pltpu.sync_copy(acc_sh.at[pl.ds(row0, RPT)],
                    acc_out.at[cidx, pl.ds(row0, RPT)])
    pltpu.sync_copy(den_sh.at[pl.ds(row0, RPT)],
                    den_out.at[pl.ds(cidx * N_ACC + row0, RPT)])


@functools.lru_cache(maxsize=None)
def _make_sc(C):
    mesh = plsc.VectorSubcoreMesh(core_axis_name="c", subcore_axis_name="s")
    return pl.kernel(
        functools.partial(_sc_body, C),
        out_type=(jax.ShapeDtypeStruct((2, N_ACC, C), jnp.float32),
                  jax.ShapeDtypeStruct((2 * N_ACC,), jnp.float32)),
        mesh=mesh,
        scratch_types=[
            pltpu.VMEM((G,), jnp.int32),
            pltpu.VMEM((G,), jnp.int32),
            pltpu.VMEM((G,), jnp.float32),
            pltpu.VMEM((G,), jnp.float32),
            pltpu.VMEM((G,), jnp.float32),
            pltpu.VMEM((G, C), jnp.float32),
            pltpu.VMEM((ZR, C), jnp.float32),
            pltpu.VMEM((RPT,), jnp.float32),
            pltpu.VMEM_SHARED((N_ACC,), jnp.float32),
            pltpu.VMEM_SHARED((N_ACC,), jnp.float32),
            pltpu.VMEM_SHARED((N_ACC, C), jnp.float32),
            pltpu.VMEM_SHARED((N_ACC,), jnp.float32),
            pltpu.SemaphoreType.DMA,
            pltpu.SemaphoreType.DMA,
        ],
        compiler_params=pltpu.CompilerParams(needs_layout_passes=False,
                                             use_tc_tiling_on_sc=False),
        name=f"gat_edge_sweep_c{C}",
    )


def _vec_spec(n):
    return pl.BlockSpec((n,), lambda i: (0,))


def _mat_spec(a, b_):
    return pl.BlockSpec((a, b_), lambda i: (0, 0))


def _first_body(x_ref, w_ref, asrc_ref, adst_ref, ha_ref, hb_ref, es_ref, ed_ref):
    h = jnp.dot(x_ref[...], w_ref[...], preferred_element_type=jnp.float32)
    ha_ref[...] = h[:, :64]
    hb_ref[...] = h[:, 64:]
    es_ref[...] = jnp.sum(h * asrc_ref[...], axis=1)[None, None, :]
    ed_ref[...] = jnp.sum(h * adst_ref[...], axis=1)[None, None, :]


_tc_first = pl.pallas_call(
    _first_body,
    grid=(B,),
    in_specs=[pl.BlockSpec((N_NODE, N_FEAT), lambda i: (i, 0)),
              _mat_spec(N_FEAT, N_FEAT), _vec_spec(N_FEAT), _vec_spec(N_FEAT)],
    out_specs=(pl.BlockSpec((N_NODE, 64), lambda i: (i, 0)),
               pl.BlockSpec((N_NODE, 64), lambda i: (i, 0)),
               pl.BlockSpec((1, 1, N_NODE), lambda i: (i, 0, 0)),
               pl.BlockSpec((1, 1, N_NODE), lambda i: (i, 0, 0))),
    out_shape=(jax.ShapeDtypeStruct((N, 64), jnp.float32),
               jax.ShapeDtypeStruct((N, 64), jnp.float32),
               jax.ShapeDtypeStruct((B, 1, N_NODE), jnp.float32),
               jax.ShapeDtypeStruct((B, 1, N_NODE), jnp.float32)),
    name="gat_first",
)


def _mid_body(nacc, acc_and_rest):
    accs = acc_and_rest[:nacc]
    (den_ref, b_ref, g_ref, o_ref, w_ref, asrc_ref, adst_ref,
     h_ref, es_ref, ed_ref) = acc_and_rest[nacc:]
    s = jnp.concatenate([a[0] + a[1] for a in (x[...] for x in accs)], axis=1)
    d = den_ref[0, 0, 0] + den_ref[1, 0, 0]
    out = s / (d + 1e-16)[:, None] + b_ref[...]
    mu = jnp.mean(out, axis=0, keepdims=True)
    var = jnp.mean((out - mu) ** 2, axis=0, keepdims=True)
    x = (out - mu) * lax.rsqrt(var + 1e-5) * g_ref[...] + o_ref[...]
    x = jnp.maximum(x, 0.0)
    h = jnp.dot(x, w_ref[...], preferred_element_type=jnp.float32)
    h_ref[...] = h
    es_ref[...] = jnp.sum(h * asrc_ref[...], axis=1)[None, None, :]
    ed_ref[...] = jnp.sum(h * adst_ref[...], axis=1)[None, None, :]


def _acc_spec(c):
    return pl.BlockSpec((2, N_NODE, c), lambda i: (0, i, 0))


_DEN_SPEC = pl.BlockSpec((2, 1, 1, N_NODE), lambda i: (0, i, 0, 0))


@functools.lru_cache(maxsize=None)
def _make_mid(nacc, cin, cout):
    def body(*refs):
        _mid_body(nacc, refs)
    cacc = cin // nacc
    return pl.pallas_call(
        body,
        grid=(B,),
        in_specs=[_acc_spec(cacc)] * nacc + [
            _DEN_SPEC, _vec_spec(cin), _vec_spec(cin), _vec_spec(cin),
            _mat_spec(cin, cout), _vec_spec(cout), _vec_spec(cout)],
        out_specs=(pl.BlockSpec((N_NODE, cout), lambda i: (i, 0)),
                   pl.BlockSpec((1, 1, N_NODE), lambda i: (i, 0, 0)),
                   pl.BlockSpec((1, 1, N_NODE), lambda i: (i, 0, 0))),
        out_shape=(jax.ShapeDtypeStruct((N, cout), jnp.float32),
                   jax.ShapeDtypeStruct((B, 1, N_NODE), jnp.float32),
                   jax.ShapeDtypeStruct((B, 1, N_NODE), jnp.float32)),
        name=f"gat_mid_{cout}",
    )


def _last_body(acc_ref, den_ref, b_ref, s_ref):
    s = acc_ref[0] + acc_ref[1]
    d = den_ref[0, 0, 0] + den_ref[1, 0, 0]
    s_ref[...] = s / (d + 1e-16)[:, None] + b_ref[...]


_tc_last = pl.pallas_call(
    _last_body,
    grid=(B,),
    in_specs=[_acc_spec(ED), _DEN_SPEC, _vec_spec(ED)],
    out_specs=pl.BlockSpec((N_NODE, ED), lambda i: (i, 0)),
    out_shape=jax.ShapeDtypeStruct((N, ED), jnp.float32),
    name="gat_last",
)


def kernel(r, W1, asrc1, adst1, b1, g1, o1, W2, asrc2, adst2, b2, g2, o2,
           W3, asrc3, adst3, b3, g3, o3, W4, asrc4, adst4, b4):
    nodes = r[:, :SLEFT].reshape(N, N_FEAT)
    off = (jnp.arange(B, dtype=jnp.int32) * N_NODE)[:, None]
    snd = (r[:, SLEFT:SLEFT + N_EDGE].astype(jnp.int32) + off).reshape(-1)
    rcv = (r[:, SLEFT + N_EDGE:SLEFT + 2 * N_EDGE].astype(jnp.int32) + off).reshape(-1)
    pad_s = (jnp.arange(PAD, dtype=jnp.int32) * 97) % N
    pad_r = N + (jnp.arange(PAD, dtype=jnp.int32) % (N_ACC - N))
    snd2 = jnp.concatenate([snd, pad_s]).reshape(NROWS, G)
    rcv2 = jnp.concatenate([rcv, pad_r]).reshape(NROWS, G)

    sc64 = _make_sc(64)
    sc32 = _make_sc(32)

    def dr(den):
        return den.reshape(2, N_ACC)[:, :N].reshape(2, B, 1, N_NODE)

    ha, hb, es, ed = _tc_first(nodes, W1, asrc1, adst1)
    es, ed = es.reshape(N), ed.reshape(N)
    acc_a, den1 = sc64(es, ed, snd2, rcv2, ha)
    acc_b, _ = sc64(es, ed, snd2, rcv2, hb)
    h2, es2, ed2 = _make_mid(2, 128, 64)(acc_a, acc_b, dr(den1),
                                         b1, g1, o1, W2, asrc2, adst2)
    acc2, den2 = sc64(es2.reshape(N), ed2.reshape(N), snd2, rcv2, h2)
    h3, es3, ed3 = _make_mid(1, 64, 64)(acc2, dr(den2),
                                        b2, g2, o2, W3, asrc3, adst3)
    acc3, den3 = sc64(es3.reshape(N), ed3.reshape(N), snd2, rcv2, h3)
    h4, es4, ed4 = _make_mid(1, 64, 32)(acc3, dr(den3),
                                        b3, g3, o3, W4, asrc4, adst4)
    acc4, den4 = sc32(es4.reshape(N), ed4.reshape(N), snd2, rcv2, h4)
    s = _tc_last(acc4, dr(den4), b4)
    return jnp.concatenate([s.reshape(B, N_NODE * ED), r[:, SLEFT:]], axis=1)


# speedup vs baseline: 35.7967x; 1.7771x over previous
import functools

import jax
import jax.numpy as jnp
from jax import lax
from jax.experimental import pallas as pl
from jax.experimental.pallas import tpu as pltpu
from jax.experimental.pallas import tpu_sc as plsc

N_NODE = 5000
N_FEAT = 128
ED = 32
MAXN = 16
N_EDGE = MAXN * (N_NODE - 1)
SLEFT = N_NODE * N_FEAT
B = 4
N = B * N_NODE
E = B * N_EDGE

G = 128
NROWS = 2560
E_PAD = NROWS * G
PAD = E_PAD - E
N_ACC = 20224
RPT = N_ACC // 16
ZR = 79
GROUPS_PER_TILE = NROWS // 32
IB = 4
NCH = GROUPS_PER_TILE // IB


def _sc_body(C, es_hbm, ed_hbm, snd_hbm, rcv_hbm, h_hbm, acc_out, den_out,
             snd_ch, rcv_ch, ex_v, esg_v, edg_v, rows_v, zbuf, dz,
             es_sh, ed_sh, acc_sh, den_sh, sem, sem2, sem3):
    cidx = lax.axis_index("c")
    sidx = lax.axis_index("s")
    row0 = sidx * RPT

    zvec = jnp.zeros((16,), jnp.float32)

    def _zb(i, carry):
        for c in range(C // 16):
            zbuf[i, pl.ds(c * 16, 16)] = zvec
        return carry
    lax.fori_loop(0, ZR, _zb, None)

    def _zd(i, carry):
        dz[pl.ds(i * 16, 16)] = zvec
        return carry
    lax.fori_loop(0, RPT // 16, _zd, None)

    for t in range(RPT // ZR):
        pltpu.sync_copy(zbuf, acc_sh.at[pl.ds(row0 + t * ZR, ZR)])
    pltpu.sync_copy(dz, den_sh.at[pl.ds(row0, RPT)])
    @pl.when(sidx == 0)
    def _stage():
        pltpu.sync_copy(es_hbm, es_sh.at[pl.ds(0, N)])
        pltpu.sync_copy(ed_hbm, ed_sh.at[pl.ds(0, N)])
        pltpu.sync_copy(dz.at[pl.ds(0, N_ACC - N)], es_sh.at[pl.ds(N, N_ACC - N)])
        pltpu.sync_copy(dz.at[pl.ds(0, N_ACC - N)], ed_sh.at[pl.ds(N, N_ACC - N)])
    plsc.subcore_barrier()

    base = cidx * (NROWS // 2) + sidx * GROUPS_PER_TILE

    pltpu.async_copy(snd_hbm.at[pl.ds(base, IB)], snd_ch.at[0], sem3)
    pltpu.async_copy(rcv_hbm.at[pl.ds(base, IB)], rcv_ch.at[0], sem3)

    def _chunk(cc, carry):
        q = lax.rem(cc, 2)
        qn = 1 - q
        pltpu.make_async_copy(snd_hbm.at[pl.ds(base, IB)], snd_ch.at[q],
                              sem3).wait()
        pltpu.make_async_copy(rcv_hbm.at[pl.ds(base, IB)], rcv_ch.at[q],
                              sem3).wait()
        nrow = base + jnp.minimum(cc + 1, NCH - 1) * IB
        pltpu.async_copy(snd_hbm.at[pl.ds(nrow, IB)], snd_ch.at[qn], sem3)
        pltpu.async_copy(rcv_hbm.at[pl.ds(nrow, IB)], rcv_ch.at[qn], sem3)
        cps = []
        for b in range(IB):
            sref = snd_ch.at[q, b]
            rref = rcv_ch.at[q, b]
            cps.append((pltpu.async_copy(h_hbm.at[sref], rows_v.at[b], sem),
                        pltpu.async_copy(es_sh.at[sref], esg_v.at[b], sem2),
                        pltpu.async_copy(ed_sh.at[rref], edg_v.at[b], sem2)))
        for b in range(IB):
            c1, c2, c3 = cps[b]
            c2.wait()
            c3.wait()
            for j in range(G // 16):
                e = esg_v[b, pl.ds(j * 16, 16)] + edg_v[b, pl.ds(j * 16, 16)]
                e = jnp.maximum(e, 0.2 * e)
                ex_v[b, pl.ds(j * 16, 16)] = jnp.exp(e)
            pltpu.sync_copy(ex_v.at[b], den_sh.at[rcv_ch.at[q, b]], add=True)
            c1.wait()

            def _scale(j, carry2, b=b):
                exv = ex_v[b, pl.ds(j * 16, 16)]
                for kk in range(16):
                    exb = jnp.broadcast_to(exv[kk], (16,))
                    k = j * 16 + kk
                    for c in range(C // 16):
                        rows_v[b, k, pl.ds(c * 16, 16)] = (
                            rows_v[b, k, pl.ds(c * 16, 16)] * exb)
                return carry2
            lax.fori_loop(0, G // 16, _scale, None)
            pltpu.sync_copy(rows_v.at[b], acc_sh.at[rcv_ch.at[q, b]], add=True)
        return carry

    lax.fori_loop(0, NCH, _chunk, None)
    pltpu.make_async_copy(snd_hbm.at[pl.ds(base, IB)], snd_ch.at[0], sem3).wait()
    pltpu.make_async_copy(rcv_hbm.at[pl.ds(base, IB)], rcv_ch.at[0], sem3).wait()
    plsc.subcore_barrier()

    pltpu.sync_copy(acc_sh.at[pl.ds(row0, RPT)],
                    acc_out.at[cidx, pl.ds(row0, RPT)])
    pltpu.sync_copy(den_sh.at[pl.ds(row0, RPT)],
                    den_out.at[pl.ds(cidx * N_ACC + row0, RPT)])


@functools.lru_cache(maxsize=None)
def _make_sc(C):
    mesh = plsc.VectorSubcoreMesh(core_axis_name="c", subcore_axis_name="s")
    return pl.kernel(
        functools.partial(_sc_body, C),
        out_type=(jax.ShapeDtypeStruct((2, N_ACC, C), jnp.float32),
                  jax.ShapeDtypeStruct((2 * N_ACC,), jnp.float32)),
        mesh=mesh,
        scratch_types=[
            pltpu.VMEM((2, IB, G), jnp.int32),
            pltpu.VMEM((2, IB, G), jnp.int32),
            pltpu.VMEM((IB, G), jnp.float32),
            pltpu.VMEM((IB, G), jnp.float32),
            pltpu.VMEM((IB, G), jnp.float32),
            pltpu.VMEM((IB, G, C), jnp.float32),
            pltpu.VMEM((ZR, C), jnp.float32),
            pltpu.VMEM((RPT,), jnp.float32),
            pltpu.VMEM_SHARED((N_ACC,), jnp.float32),
            pltpu.VMEM_SHARED((N_ACC,), jnp.float32),
            pltpu.VMEM_SHARED((N_ACC, C), jnp.float32),
            pltpu.VMEM_SHARED((N_ACC,), jnp.float32),
            pltpu.SemaphoreType.DMA,
            pltpu.SemaphoreType.DMA,
            pltpu.SemaphoreType.DMA,
        ],
        compiler_params=pltpu.CompilerParams(needs_layout_passes=False,
                                             use_tc_tiling_on_sc=False),
        name=f"gat_edge_sweep_c{C}",
    )


def _vec_spec(n):
    return pl.BlockSpec((n,), lambda i: (0,))


def _mat_spec(a, b_):
    return pl.BlockSpec((a, b_), lambda i: (0, 0))


def _first_body(x_ref, w_ref, asrc_ref, adst_ref, ha_ref, hb_ref, es_ref, ed_ref):
    h = jnp.dot(x_ref[...], w_ref[...], preferred_element_type=jnp.float32)
    ha_ref[...] = h[:, :64]
    hb_ref[...] = h[:, 64:]
    es_ref[...] = jnp.sum(h * asrc_ref[...], axis=1)[None, None, :]
    ed_ref[...] = jnp.sum(h * adst_ref[...], axis=1)[None, None, :]


_tc_first = pl.pallas_call(
    _first_body,
    grid=(B,),
    in_specs=[pl.BlockSpec((N_NODE, N_FEAT), lambda i: (i, 0)),
              _mat_spec(N_FEAT, N_FEAT), _vec_spec(N_FEAT), _vec_spec(N_FEAT)],
    out_specs=(pl.BlockSpec((N_NODE, 64), lambda i: (i, 0)),
               pl.BlockSpec((N_NODE, 64), lambda i: (i, 0)),
               pl.BlockSpec((1, 1, N_NODE), lambda i: (i, 0, 0)),
               pl.BlockSpec((1, 1, N_NODE), lambda i: (i, 0, 0))),
    out_shape=(jax.ShapeDtypeStruct((N, 64), jnp.float32),
               jax.ShapeDtypeStruct((N, 64), jnp.float32),
               jax.ShapeDtypeStruct((B, 1, N_NODE), jnp.float32),
               jax.ShapeDtypeStruct((B, 1, N_NODE), jnp.float32)),
    name="gat_first",
)


def _mid_body(nacc, acc_and_rest):
    accs = acc_and_rest[:nacc]
    (den_ref, b_ref, g_ref, o_ref, w_ref, asrc_ref, adst_ref,
     h_ref, es_ref, ed_ref) = acc_and_rest[nacc:]
    s = jnp.concatenate([a[0] + a[1] for a in (x[...] for x in accs)], axis=1)
    d = den_ref[0, 0, 0] + den_ref[1, 0, 0]
    out = s / (d + 1e-16)[:, None] + b_ref[...]
    mu = jnp.mean(out, axis=0, keepdims=True)
    var = jnp.mean((out - mu) ** 2, axis=0, keepdims=True)
    x = (out - mu) * lax.rsqrt(var + 1e-5) * g_ref[...] + o_ref[...]
    x = jnp.maximum(x, 0.0)
    h = jnp.dot(x, w_ref[...], preferred_element_type=jnp.float32)
    h_ref[...] = h
    es_ref[...] = jnp.sum(h * asrc_ref[...], axis=1)[None, None, :]
    ed_ref[...] = jnp.sum(h * adst_ref[...], axis=1)[None, None, :]


def _acc_spec(c):
    return pl.BlockSpec((2, N_NODE, c), lambda i: (0, i, 0))


_DEN_SPEC = pl.BlockSpec((2, 1, 1, N_NODE), lambda i: (0, i, 0, 0))


@functools.lru_cache(maxsize=None)
def _make_mid(nacc, cin, cout):
    def body(*refs):
        _mid_body(nacc, refs)
    cacc = cin // nacc
    return pl.pallas_call(
        body,
        grid=(B,),
        in_specs=[_acc_spec(cacc)] * nacc + [
            _DEN_SPEC, _vec_spec(cin), _vec_spec(cin), _vec_spec(cin),
            _mat_spec(cin, cout), _vec_spec(cout), _vec_spec(cout)],
        out_specs=(pl.BlockSpec((N_NODE, cout), lambda i: (i, 0)),
                   pl.BlockSpec((1, 1, N_NODE), lambda i: (i, 0, 0)),
                   pl.BlockSpec((1, 1, N_NODE), lambda i: (i, 0, 0))),
        out_shape=(jax.ShapeDtypeStruct((N, cout), jnp.float32),
                   jax.ShapeDtypeStruct((B, 1, N_NODE), jnp.float32),
                   jax.ShapeDtypeStruct((B, 1, N_NODE), jnp.float32)),
        name=f"gat_mid_{cout}",
    )


def _last_body(acc_ref, den_ref, b_ref, s_ref):
    s = acc_ref[0] + acc_ref[1]
    d = den_ref[0, 0, 0] + den_ref[1, 0, 0]
    s_ref[...] = s / (d + 1e-16)[:, None] + b_ref[...]


_tc_last = pl.pallas_call(
    _last_body,
    grid=(B,),
    in_specs=[_acc_spec(ED), _DEN_SPEC, _vec_spec(ED)],
    out_specs=pl.BlockSpec((N_NODE, ED), lambda i: (i, 0)),
    out_shape=jax.ShapeDtypeStruct((N, ED), jnp.float32),
    name="gat_last",
)


def kernel(r, W1, asrc1, adst1, b1, g1, o1, W2, asrc2, adst2, b2, g2, o2,
           W3, asrc3, adst3, b3, g3, o3, W4, asrc4, adst4, b4):
    nodes = r[:, :SLEFT].reshape(N, N_FEAT)
    off = (jnp.arange(B, dtype=jnp.int32) * N_NODE)[:, None]
    snd = (r[:, SLEFT:SLEFT + N_EDGE].astype(jnp.int32) + off).reshape(-1)
    rcv = (r[:, SLEFT + N_EDGE:SLEFT + 2 * N_EDGE].astype(jnp.int32) + off).reshape(-1)
    pad_s = (jnp.arange(PAD, dtype=jnp.int32) * 97) % N
    pad_r = N + (jnp.arange(PAD, dtype=jnp.int32) % (N_ACC - N))
    snd2 = jnp.concatenate([snd, pad_s]).reshape(NROWS, G)
    rcv2 = jnp.concatenate([rcv, pad_r]).reshape(NROWS, G)

    sc64 = _make_sc(64)
    sc32 = _make_sc(32)

    def dr(den):
        return den.reshape(2, N_ACC)[:, :N].reshape(2, B, 1, N_NODE)

    ha, hb, es, ed = _tc_first(nodes, W1, asrc1, adst1)
    es, ed = es.reshape(N), ed.reshape(N)
    acc_a, den1 = sc64(es, ed, snd2, rcv2, ha)
    acc_b, _ = sc64(es, ed, snd2, rcv2, hb)
    h2, es2, ed2 = _make_mid(2, 128, 64)(acc_a, acc_b, dr(den1),
                                         b1, g1, o1, W2, asrc2, adst2)
    acc2, den2 = sc64(es2.reshape(N), ed2.reshape(N), snd2, rcv2, h2)
    h3, es3, ed3 = _make_mid(1, 64, 64)(acc2, dr(den2),
                                        b2, g2, o2, W3, asrc3, adst3)
    acc3, den3 = sc64(es3.reshape(N), ed3.reshape(N), snd2, rcv2, h3)
    h4, es4, ed4 = _make_mid(1, 64, 32)(acc3, dr(den3),
                                        b3, g3, o3, W4, asrc4, adst4)
    acc4, den4 = sc32(es4.reshape(N), ed4.reshape(N), snd2, rcv2, h4)
    s = _tc_last(acc4, dr(den4), b4)
    return jnp.concatenate([s.reshape(B, N_NODE * ED), r[:, SLEFT:]], axis=1)


# speedup vs baseline: 39.8264x; 1.1126x over previous
import functools

import jax
import jax.numpy as jnp
from jax import lax
from jax.experimental import pallas as pl
from jax.experimental.pallas import tpu as pltpu
from jax.experimental.pallas import tpu_sc as plsc

N_NODE = 5000
N_FEAT = 128
ED = 32
MAXN = 16
N_EDGE = MAXN * (N_NODE - 1)
SLEFT = N_NODE * N_FEAT
B = 4
N = B * N_NODE
E = B * N_EDGE

G = 128
NROWS = 2560
E_PAD = NROWS * G
PAD = E_PAD - E
N_ACC = 20224
RPT = N_ACC // 16
ZR = 79
GROUPS_PER_TILE = NROWS // 32
IB = 4
NCH = GROUPS_PER_TILE // IB


def _sc_body(C, es_hbm, ed_hbm, snd_hbm, rcv_hbm, h_hbm, acc_out, den_out,
             snd_ch, rcv_ch, ex_v, esg_v, edg_v, rows_v, zbuf, dz,
             es_sh, ed_sh, acc_sh, den_sh, sem, sem2, sem3, sem4, sem5):
    cidx = lax.axis_index("c")
    sidx = lax.axis_index("s")
    row0 = sidx * RPT

    zvec = jnp.zeros((16,), jnp.float32)

    def _zb(i, carry):
        for c in range(C // 16):
            zbuf[i, pl.ds(c * 16, 16)] = zvec
        return carry
    lax.fori_loop(0, ZR, _zb, None)

    def _zd(i, carry):
        dz[pl.ds(i * 16, 16)] = zvec
        return carry
    lax.fori_loop(0, RPT // 16, _zd, None)

    for t in range(RPT // ZR):
        pltpu.sync_copy(zbuf, acc_sh.at[pl.ds(row0 + t * ZR, ZR)])
    pltpu.sync_copy(dz, den_sh.at[pl.ds(row0, RPT)])
    @pl.when(sidx == 0)
    def _stage():
        pltpu.sync_copy(es_hbm, es_sh.at[pl.ds(0, N)])
        pltpu.sync_copy(ed_hbm, ed_sh.at[pl.ds(0, N)])
        pltpu.sync_copy(dz.at[pl.ds(0, N_ACC - N)], es_sh.at[pl.ds(N, N_ACC - N)])
        pltpu.sync_copy(dz.at[pl.ds(0, N_ACC - N)], ed_sh.at[pl.ds(N, N_ACC - N)])
    plsc.subcore_barrier()

    base = cidx * (NROWS // 2) + sidx * GROUPS_PER_TILE

    pltpu.async_copy(snd_hbm.at[pl.ds(base, IB)], snd_ch.at[0], sem3)
    pltpu.async_copy(rcv_hbm.at[pl.ds(base, IB)], rcv_ch.at[0], sem3)

    def _chunk(cc, carry):
        q = lax.rem(cc, 2)
        qn = 1 - q
        @pl.when(cc > 0)
        def _drain():
            for b in range(IB):
                pltpu.make_async_copy(ex_v.at[b], den_sh.at[rcv_ch.at[q, b]],
                                      sem4).wait()
                pltpu.make_async_copy(rows_v.at[b], acc_sh.at[rcv_ch.at[q, b]],
                                      sem5).wait()
        pltpu.make_async_copy(snd_hbm.at[pl.ds(base, IB)], snd_ch.at[q],
                              sem3).wait()
        pltpu.make_async_copy(rcv_hbm.at[pl.ds(base, IB)], rcv_ch.at[q],
                              sem3).wait()
        nrow = base + jnp.minimum(cc + 1, NCH - 1) * IB
        pltpu.async_copy(snd_hbm.at[pl.ds(nrow, IB)], snd_ch.at[qn], sem3)
        pltpu.async_copy(rcv_hbm.at[pl.ds(nrow, IB)], rcv_ch.at[qn], sem3)
        cps = []
        for b in range(IB):
            sref = snd_ch.at[q, b]
            rref = rcv_ch.at[q, b]
            cps.append((pltpu.async_copy(h_hbm.at[sref], rows_v.at[b], sem),
                        pltpu.async_copy(es_sh.at[sref], esg_v.at[b], sem2),
                        pltpu.async_copy(ed_sh.at[rref], edg_v.at[b], sem2)))
        for b in range(IB):
            c1, c2, c3 = cps[b]
            c2.wait()
            c3.wait()
            for j in range(G // 16):
                e = esg_v[b, pl.ds(j * 16, 16)] + edg_v[b, pl.ds(j * 16, 16)]
                e = jnp.maximum(e, 0.2 * e)
                ex_v[b, pl.ds(j * 16, 16)] = jnp.exp(e)
            pltpu.async_copy(ex_v.at[b], den_sh.at[rcv_ch.at[q, b]], sem4,
                             add=True)
            c1.wait()

            def _scale(j, carry2, b=b):
                exv = ex_v[b, pl.ds(j * 16, 16)]
                for kk in range(16):
                    exb = jnp.broadcast_to(exv[kk], (16,))
                    k = j * 16 + kk
                    for c in range(C // 16):
                        rows_v[b, k, pl.ds(c * 16, 16)] = (
                            rows_v[b, k, pl.ds(c * 16, 16)] * exb)
                return carry2
            lax.fori_loop(0, G // 16, _scale, None)
            pltpu.async_copy(rows_v.at[b], acc_sh.at[rcv_ch.at[q, b]], sem5,
                             add=True)
        return carry

    lax.fori_loop(0, NCH, _chunk, None)
    for b in range(IB):
        pltpu.make_async_copy(ex_v.at[b], den_sh.at[rcv_ch.at[0, b]],
                              sem4).wait()
        pltpu.make_async_copy(rows_v.at[b], acc_sh.at[rcv_ch.at[0, b]],
                              sem5).wait()
    pltpu.make_async_copy(snd_hbm.at[pl.ds(base, IB)], snd_ch.at[0], sem3).wait()
    pltpu.make_async_copy(rcv_hbm.at[pl.ds(base, IB)], rcv_ch.at[0], sem3).wait()
    plsc.subcore_barrier()

    pltpu.sync_copy(acc_sh.at[pl.ds(row0, RPT)],
                    acc_out.at[cidx, pl.ds(row0, RPT)])
    pltpu.sync_copy(den_sh.at[pl.ds(row0, RPT)],
                    den_out.at[pl.ds(cidx * N_ACC + row0, RPT)])


@functools.lru_cache(maxsize=None)
def _make_sc(C):
    mesh = plsc.VectorSubcoreMesh(core_axis_name="c", subcore_axis_name="s")
    return pl.kernel(
        functools.partial(_sc_body, C),
        out_type=(jax.ShapeDtypeStruct((2, N_ACC, C), jnp.float32),
                  jax.ShapeDtypeStruct((2 * N_ACC,), jnp.float32)),
        mesh=mesh,
        scratch_types=[
            pltpu.VMEM((2, IB, G), jnp.int32),
            pltpu.VMEM((2, IB, G), jnp.int32),
            pltpu.VMEM((IB, G), jnp.float32),
            pltpu.VMEM((IB, G), jnp.float32),
            pltpu.VMEM((IB, G), jnp.float32),
            pltpu.VMEM((IB, G, C), jnp.float32),
            pltpu.VMEM((ZR, C), jnp.float32),
            pltpu.VMEM((RPT,), jnp.float32),
            pltpu.VMEM_SHARED((N_ACC,), jnp.float32),
            pltpu.VMEM_SHARED((N_ACC,), jnp.float32),
            pltpu.VMEM_SHARED((N_ACC, C), jnp.float32),
            pltpu.VMEM_SHARED((N_ACC,), jnp.float32),
            pltpu.SemaphoreType.DMA,
            pltpu.SemaphoreType.DMA,
            pltpu.SemaphoreType.DMA,
            pltpu.SemaphoreType.DMA,
            pltpu.SemaphoreType.DMA,
        ],
        compiler_params=pltpu.CompilerParams(needs_layout_passes=False,
                                             use_tc_tiling_on_sc=False),
        name=f"gat_edge_sweep_c{C}",
    )


def _vec_spec(n):
    return pl.BlockSpec((n,), lambda i: (0,))


def _mat_spec(a, b_):
    return pl.BlockSpec((a, b_), lambda i: (0, 0))


def _first_body(x_ref, w_ref, asrc_ref, adst_ref, ha_ref, hb_ref, es_ref, ed_ref):
    h = jnp.dot(x_ref[...], w_ref[...], preferred_element_type=jnp.float32)
    ha_ref[...] = h[:, :64]
    hb_ref[...] = h[:, 64:]
    es_ref[...] = jnp.sum(h * asrc_ref[...], axis=1)[None, None, :]
    ed_ref[...] = jnp.sum(h * adst_ref[...], axis=1)[None, None, :]


_tc_first = pl.pallas_call(
    _first_body,
    grid=(B,),
    in_specs=[pl.BlockSpec((N_NODE, N_FEAT), lambda i: (i, 0)),
              _mat_spec(N_FEAT, N_FEAT), _vec_spec(N_FEAT), _vec_spec(N_FEAT)],
    out_specs=(pl.BlockSpec((N_NODE, 64), lambda i: (i, 0)),
               pl.BlockSpec((N_NODE, 64), lambda i: (i, 0)),
               pl.BlockSpec((1, 1, N_NODE), lambda i: (i, 0, 0)),
               pl.BlockSpec((1, 1, N_NODE), lambda i: (i, 0, 0))),
    out_shape=(jax.ShapeDtypeStruct((N, 64), jnp.float32),
               jax.ShapeDtypeStruct((N, 64), jnp.float32),
               jax.ShapeDtypeStruct((B, 1, N_NODE), jnp.float32),
               jax.ShapeDtypeStruct((B, 1, N_NODE), jnp.float32)),
    name="gat_first",
)


def _mid_body(nacc, acc_and_rest):
    accs = acc_and_rest[:nacc]
    (den_ref, b_ref, g_ref, o_ref, w_ref, asrc_ref, adst_ref,
     h_ref, es_ref, ed_ref) = acc_and_rest[nacc:]
    s = jnp.concatenate([a[0] + a[1] for a in (x[...] for x in accs)], axis=1)
    d = den_ref[0, 0, 0] + den_ref[1, 0, 0]
    out = s / (d + 1e-16)[:, None] + b_ref[...]
    mu = jnp.mean(out, axis=0, keepdims=True)
    var = jnp.mean((out - mu) ** 2, axis=0, keepdims=True)
    x = (out - mu) * lax.rsqrt(var + 1e-5) * g_ref[...] + o_ref[...]
    x = jnp.maximum(x, 0.0)
    h = jnp.dot(x, w_ref[...], preferred_element_type=jnp.float32)
    h_ref[...] = h
    es_ref[...] = jnp.sum(h * asrc_ref[...], axis=1)[None, None, :]
    ed_ref[...] = jnp.sum(h * adst_ref[...], axis=1)[None, None, :]


def _acc_spec(c):
    return pl.BlockSpec((2, N_NODE, c), lambda i: (0, i, 0))


_DEN_SPEC = pl.BlockSpec((2, 1, 1, N_NODE), lambda i: (0, i, 0, 0))


@functools.lru_cache(maxsize=None)
def _make_mid(nacc, cin, cout):
    def body(*refs):
        _mid_body(nacc, refs)
    cacc = cin // nacc
    return pl.pallas_call(
        body,
        grid=(B,),
        in_specs=[_acc_spec(cacc)] * nacc + [
            _DEN_SPEC, _vec_spec(cin), _vec_spec(cin), _vec_spec(cin),
            _mat_spec(cin, cout), _vec_spec(cout), _vec_spec(cout)],
        out_specs=(pl.BlockSpec((N_NODE, cout), lambda i: (i, 0)),
                   pl.BlockSpec((1, 1, N_NODE), lambda i: (i, 0, 0)),
                   pl.BlockSpec((1, 1, N_NODE), lambda i: (i, 0, 0))),
        out_shape=(jax.ShapeDtypeStruct((N, cout), jnp.float32),
                   jax.ShapeDtypeStruct((B, 1, N_NODE), jnp.float32),
                   jax.ShapeDtypeStruct((B, 1, N_NODE), jnp.float32)),
        name=f"gat_mid_{cout}",
    )


def _last_body(acc_ref, den_ref, b_ref, s_ref):
    s = acc_ref[0] + acc_ref[1]
    d = den_ref[0, 0, 0] + den_ref[1, 0, 0]
    s_ref[...] = s / (d + 1e-16)[:, None] + b_ref[...]


_tc_last = pl.pallas_call(
    _last_body,
    grid=(B,),
    in_specs=[_acc_spec(ED), _DEN_SPEC, _vec_spec(ED)],
    out_specs=pl.BlockSpec((N_NODE, ED), lambda i: (i, 0)),
    out_shape=jax.ShapeDtypeStruct((N, ED), jnp.float32),
    name="gat_last",
)


def kernel(r, W1, asrc1, adst1, b1, g1, o1, W2, asrc2, adst2, b2, g2, o2,
           W3, asrc3, adst3, b3, g3, o3, W4, asrc4, adst4, b4):
    nodes = r[:, :SLEFT].reshape(N, N_FEAT)
    off = (jnp.arange(B, dtype=jnp.int32) * N_NODE)[:, None]
    snd = (r[:, SLEFT:SLEFT + N_EDGE].astype(jnp.int32) + off).reshape(-1)
    rcv = (r[:, SLEFT + N_EDGE:SLEFT + 2 * N_EDGE].astype(jnp.int32) + off).reshape(-1)
    pad_s = (jnp.arange(PAD, dtype=jnp.int32) * 97) % N
    pad_r = N + (jnp.arange(PAD, dtype=jnp.int32) % (N_ACC - N))
    snd2 = jnp.concatenate([snd, pad_s]).reshape(NROWS, G)
    rcv2 = jnp.concatenate([rcv, pad_r]).reshape(NROWS, G)

    sc64 = _make_sc(64)
    sc32 = _make_sc(32)

    def dr(den):
        return den.reshape(2, N_ACC)[:, :N].reshape(2, B, 1, N_NODE)

    ha, hb, es, ed = _tc_first(nodes, W1, asrc1, adst1)
    es, ed = es.reshape(N), ed.reshape(N)
    acc_a, den1 = sc64(es, ed, snd2, rcv2, ha)
    acc_b, _ = sc64(es, ed, snd2, rcv2, hb)
    h2, es2, ed2 = _make_mid(2, 128, 64)(acc_a, acc_b, dr(den1),
                                         b1, g1, o1, W2, asrc2, adst2)
    acc2, den2 = sc64(es2.reshape(N), ed2.reshape(N), snd2, rcv2, h2)
    h3, es3, ed3 = _make_mid(1, 64, 64)(acc2, dr(den2),
                                        b2, g2, o2, W3, asrc3, adst3)
    acc3, den3 = sc64(es3.reshape(N), ed3.reshape(N), snd2, rcv2, h3)
    h4, es4, ed4 = _make_mid(1, 64, 32)(acc3, dr(den3),
                                        b3, g3, o3, W4, asrc4, adst4)
    acc4, den4 = sc32(es4.reshape(N), ed4.reshape(N), snd2, rcv2, h4)
    s = _tc_last(acc4, dr(den4), b4)
    return jnp.concatenate([s.reshape(B, N_NODE * ED), r[:, SLEFT:]], axis=1)


# speedup vs baseline: 48.3351x; 1.2136x over previous
import functools

import jax
import jax.numpy as jnp
from jax import lax
from jax.experimental import pallas as pl
from jax.experimental.pallas import tpu as pltpu
from jax.experimental.pallas import tpu_sc as plsc

N_NODE = 5000
N_FEAT = 128
ED = 32
MAXN = 16
N_EDGE = MAXN * (N_NODE - 1)
SLEFT = N_NODE * N_FEAT
B = 4
N = B * N_NODE
E = B * N_EDGE

G = 128
NROWS = 2560
E_PAD = NROWS * G
PAD = E_PAD - E
N_ACC = 20224
RPT = N_ACC // 16
ZR = 79
GROUPS_PER_TILE = NROWS // 32
IB = 2
NCH = GROUPS_PER_TILE // IB


def _sc_body(C, es_hbm, ed_hbm, snd_hbm, rcv_hbm, h_hbm, acc_out, den_out,
             snd_ch, rcv_ch, ex_v, esg_v, edg_v, rows_v, zbuf, dz,
             es_sh, ed_sh, acc_sh, den_sh, sem, sem2, sem3, sem4, sem5):
    cidx = lax.axis_index("c")
    sidx = lax.axis_index("s")
    row0 = sidx * RPT

    zvec = jnp.zeros((16,), jnp.float32)

    def _zb(i, carry):
        for c in range(C // 16):
            zbuf[i, pl.ds(c * 16, 16)] = zvec
        return carry
    lax.fori_loop(0, ZR, _zb, None)

    def _zd(i, carry):
        dz[pl.ds(i * 16, 16)] = zvec
        return carry
    lax.fori_loop(0, RPT // 16, _zd, None)

    for t in range(RPT // ZR):
        pltpu.sync_copy(zbuf, acc_sh.at[pl.ds(row0 + t * ZR, ZR)])
    pltpu.sync_copy(dz, den_sh.at[pl.ds(row0, RPT)])
    @pl.when(sidx == 0)
    def _stage():
        pltpu.sync_copy(es_hbm, es_sh.at[pl.ds(0, N)])
        pltpu.sync_copy(ed_hbm, ed_sh.at[pl.ds(0, N)])
        pltpu.sync_copy(dz.at[pl.ds(0, N_ACC - N)], es_sh.at[pl.ds(N, N_ACC - N)])
        pltpu.sync_copy(dz.at[pl.ds(0, N_ACC - N)], ed_sh.at[pl.ds(N, N_ACC - N)])
    plsc.subcore_barrier()

    base = cidx * (NROWS // 2) + sidx * GROUPS_PER_TILE

    pltpu.async_copy(snd_hbm.at[pl.ds(base, IB)], snd_ch.at[0], sem3)
    pltpu.async_copy(rcv_hbm.at[pl.ds(base, IB)], rcv_ch.at[0], sem3)

    def _chunk(cc, carry):
        q = lax.rem(cc, 2)
        qn = 1 - q
        @pl.when(cc > 0)
        def _drain():
            for b in range(IB):
                pltpu.make_async_copy(ex_v.at[b], den_sh.at[rcv_ch.at[q, b]],
                                      sem4).wait()
                pltpu.make_async_copy(rows_v.at[b], acc_sh.at[rcv_ch.at[q, b]],
                                      sem5).wait()
        pltpu.make_async_copy(snd_hbm.at[pl.ds(base, IB)], snd_ch.at[q],
                              sem3).wait()
        pltpu.make_async_copy(rcv_hbm.at[pl.ds(base, IB)], rcv_ch.at[q],
                              sem3).wait()
        nrow = base + jnp.minimum(cc + 1, NCH - 1) * IB
        pltpu.async_copy(snd_hbm.at[pl.ds(nrow, IB)], snd_ch.at[qn], sem3)
        pltpu.async_copy(rcv_hbm.at[pl.ds(nrow, IB)], rcv_ch.at[qn], sem3)
        cps = []
        for b in range(IB):
            sref = snd_ch.at[q, b]
            rref = rcv_ch.at[q, b]
            cps.append((pltpu.async_copy(h_hbm.at[sref], rows_v.at[b], sem),
                        pltpu.async_copy(es_sh.at[sref], esg_v.at[b], sem2),
                        pltpu.async_copy(ed_sh.at[rref], edg_v.at[b], sem2)))
        for b in range(IB):
            c1, c2, c3 = cps[b]
            c2.wait()
            c3.wait()
            for j in range(G // 16):
                e = esg_v[b, pl.ds(j * 16, 16)] + edg_v[b, pl.ds(j * 16, 16)]
                e = jnp.maximum(e, 0.2 * e)
                ex_v[b, pl.ds(j * 16, 16)] = jnp.exp(e)
            pltpu.async_copy(ex_v.at[b], den_sh.at[rcv_ch.at[q, b]], sem4,
                             add=True)
            c1.wait()

            for j in range(G // 16):
                exv = ex_v[b, pl.ds(j * 16, 16)]
                for kk in range(16):
                    exb = jnp.broadcast_to(exv[kk], (16,))
                    k = j * 16 + kk
                    for c in range(C // 16):
                        rows_v[b, k, pl.ds(c * 16, 16)] = (
                            rows_v[b, k, pl.ds(c * 16, 16)] * exb)
            pltpu.async_copy(rows_v.at[b], acc_sh.at[rcv_ch.at[q, b]], sem5,
                             add=True)
        return carry

    lax.fori_loop(0, NCH, _chunk, None)
    for b in range(IB):
        pltpu.make_async_copy(ex_v.at[b], den_sh.at[rcv_ch.at[0, b]],
                              sem4).wait()
        pltpu.make_async_copy(rows_v.at[b], acc_sh.at[rcv_ch.at[0, b]],
                              sem5).wait()
    pltpu.make_async_copy(snd_hbm.at[pl.ds(base, IB)], snd_ch.at[0], sem3).wait()
    pltpu.make_async_copy(rcv_hbm.at[pl.ds(base, IB)], rcv_ch.at[0], sem3).wait()
    plsc.subcore_barrier()

    pltpu.sync_copy(acc_sh.at[pl.ds(row0, RPT)],
                    acc_out.at[cidx, pl.ds(row0, RPT)])
    pltpu.sync_copy(den_sh.at[pl.ds(row0, RPT)],
                    den_out.at[pl.ds(cidx * N_ACC + row0, RPT)])


@functools.lru_cache(maxsize=None)
def _make_sc(C):
    mesh = plsc.VectorSubcoreMesh(core_axis_name="c", subcore_axis_name="s")
    return pl.kernel(
        functools.partial(_sc_body, C),
        out_type=(jax.ShapeDtypeStruct((2, N_ACC, C), jnp.float32),
                  jax.ShapeDtypeStruct((2 * N_ACC,), jnp.float32)),
        mesh=mesh,
        scratch_types=[
            pltpu.VMEM((2, IB, G), jnp.int32),
            pltpu.VMEM((2, IB, G), jnp.int32),
            pltpu.VMEM((IB, G), jnp.float32),
            pltpu.VMEM((IB, G), jnp.float32),
            pltpu.VMEM((IB, G), jnp.float32),
            pltpu.VMEM((IB, G, C), jnp.float32),
            pltpu.VMEM((ZR, C), jnp.float32),
            pltpu.VMEM((RPT,), jnp.float32),
            pltpu.VMEM_SHARED((N_ACC,), jnp.float32),
            pltpu.VMEM_SHARED((N_ACC,), jnp.float32),
            pltpu.VMEM_SHARED((N_ACC, C), jnp.float32),
            pltpu.VMEM_SHARED((N_ACC,), jnp.float32),
            pltpu.SemaphoreType.DMA,
            pltpu.SemaphoreType.DMA,
            pltpu.SemaphoreType.DMA,
            pltpu.SemaphoreType.DMA,
            pltpu.SemaphoreType.DMA,
        ],
        compiler_params=pltpu.CompilerParams(needs_layout_passes=False,
                                             use_tc_tiling_on_sc=False),
        name=f"gat_edge_sweep_c{C}",
    )


def _vec_spec(n):
    return pl.BlockSpec((n,), lambda i: (0,))


def _mat_spec(a, b_):
    return pl.BlockSpec((a, b_), lambda i: (0, 0))


def _first_body(x_ref, w_ref, asrc_ref, adst_ref, ha_ref, hb_ref, es_ref, ed_ref):
    h = jnp.dot(x_ref[...], w_ref[...], preferred_element_type=jnp.float32)
    ha_ref[...] = h[:, :64]
    hb_ref[...] = h[:, 64:]
    es_ref[...] = jnp.sum(h * asrc_ref[...], axis=1)[None, None, :]
    ed_ref[...] = jnp.sum(h * adst_ref[...], axis=1)[None, None, :]


_tc_first = pl.pallas_call(
    _first_body,
    grid=(B,),
    in_specs=[pl.BlockSpec((N_NODE, N_FEAT), lambda i: (i, 0)),
              _mat_spec(N_FEAT, N_FEAT), _vec_spec(N_FEAT), _vec_spec(N_FEAT)],
    out_specs=(pl.BlockSpec((N_NODE, 64), lambda i: (i, 0)),
               pl.BlockSpec((N_NODE, 64), lambda i: (i, 0)),
               pl.BlockSpec((1, 1, N_NODE), lambda i: (i, 0, 0)),
               pl.BlockSpec((1, 1, N_NODE), lambda i: (i, 0, 0))),
    out_shape=(jax.ShapeDtypeStruct((N, 64), jnp.float32),
               jax.ShapeDtypeStruct((N, 64), jnp.float32),
               jax.ShapeDtypeStruct((B, 1, N_NODE), jnp.float32),
               jax.ShapeDtypeStruct((B, 1, N_NODE), jnp.float32)),
    name="gat_first",
)


def _mid_body(nacc, acc_and_rest):
    accs = acc_and_rest[:nacc]
    (den_ref, b_ref, g_ref, o_ref, w_ref, asrc_ref, adst_ref,
     h_ref, es_ref, ed_ref) = acc_and_rest[nacc:]
    s = jnp.concatenate([a[0] + a[1] for a in (x[...] for x in accs)], axis=1)
    d = den_ref[0, 0, 0] + den_ref[1, 0, 0]
    out = s / (d + 1e-16)[:, None] + b_ref[...]
    mu = jnp.mean(out, axis=0, keepdims=True)
    var = jnp.mean((out - mu) ** 2, axis=0, keepdims=True)
    x = (out - mu) * lax.rsqrt(var + 1e-5) * g_ref[...] + o_ref[...]
    x = jnp.maximum(x, 0.0)
    h = jnp.dot(x, w_ref[...], preferred_element_type=jnp.float32)
    h_ref[...] = h
    es_ref[...] = jnp.sum(h * asrc_ref[...], axis=1)[None, None, :]
    ed_ref[...] = jnp.sum(h * adst_ref[...], axis=1)[None, None, :]


def _acc_spec(c):
    return pl.BlockSpec((2, N_NODE, c), lambda i: (0, i, 0))


_DEN_SPEC = pl.BlockSpec((2, 1, 1, N_NODE), lambda i: (0, i, 0, 0))


@functools.lru_cache(maxsize=None)
def _make_mid(nacc, cin, cout):
    def body(*refs):
        _mid_body(nacc, refs)
    cacc = cin // nacc
    return pl.pallas_call(
        body,
        grid=(B,),
        in_specs=[_acc_spec(cacc)] * nacc + [
            _DEN_SPEC, _vec_spec(cin), _vec_spec(cin), _vec_spec(cin),
            _mat_spec(cin, cout), _vec_spec(cout), _vec_spec(cout)],
        out_specs=(pl.BlockSpec((N_NODE, cout), lambda i: (i, 0)),
                   pl.BlockSpec((1, 1, N_NODE), lambda i: (i, 0, 0)),
                   pl.BlockSpec((1, 1, N_NODE), lambda i: (i, 0, 0))),
        out_shape=(jax.ShapeDtypeStruct((N, cout), jnp.float32),
                   jax.ShapeDtypeStruct((B, 1, N_NODE), jnp.float32),
                   jax.ShapeDtypeStruct((B, 1, N_NODE), jnp.float32)),
        name=f"gat_mid_{cout}",
    )


def _last_body(acc_ref, den_ref, b_ref, s_ref):
    s = acc_ref[0] + acc_ref[1]
    d = den_ref[0, 0, 0] + den_ref[1, 0, 0]
    s_ref[...] = s / (d + 1e-16)[:, None] + b_ref[...]


_tc_last = pl.pallas_call(
    _last_body,
    grid=(B,),
    in_specs=[_acc_spec(ED), _DEN_SPEC, _vec_spec(ED)],
    out_specs=pl.BlockSpec((N_NODE, ED), lambda i: (i, 0)),
    out_shape=jax.ShapeDtypeStruct((N, ED), jnp.float32),
    name="gat_last",
)


def kernel(r, W1, asrc1, adst1, b1, g1, o1, W2, asrc2, adst2, b2, g2, o2,
           W3, asrc3, adst3, b3, g3, o3, W4, asrc4, adst4, b4):
    nodes = r[:, :SLEFT].reshape(N, N_FEAT)
    off = (jnp.arange(B, dtype=jnp.int32) * N_NODE)[:, None]
    snd = (r[:, SLEFT:SLEFT + N_EDGE].astype(jnp.int32) + off).reshape(-1)
    rcv = (r[:, SLEFT + N_EDGE:SLEFT + 2 * N_EDGE].astype(jnp.int32) + off).reshape(-1)
    pad_s = (jnp.arange(PAD, dtype=jnp.int32) * 97) % N
    pad_r = N + (jnp.arange(PAD, dtype=jnp.int32) % (N_ACC - N))
    snd2 = jnp.concatenate([snd, pad_s]).reshape(NROWS, G)
    rcv2 = jnp.concatenate([rcv, pad_r]).reshape(NROWS, G)

    sc64 = _make_sc(64)
    sc32 = _make_sc(32)

    def dr(den):
        return den.reshape(2, N_ACC)[:, :N].reshape(2, B, 1, N_NODE)

    ha, hb, es, ed = _tc_first(nodes, W1, asrc1, adst1)
    es, ed = es.reshape(N), ed.reshape(N)
    acc_a, den1 = sc64(es, ed, snd2, rcv2, ha)
    acc_b, _ = sc64(es, ed, snd2, rcv2, hb)
    h2, es2, ed2 = _make_mid(2, 128, 64)(acc_a, acc_b, dr(den1),
                                         b1, g1, o1, W2, asrc2, adst2)
    acc2, den2 = sc64(es2.reshape(N), ed2.reshape(N), snd2, rcv2, h2)
    h3, es3, ed3 = _make_mid(1, 64, 64)(acc2, dr(den2),
                                        b2, g2, o2, W3, asrc3, adst3)
    acc3, den3 = sc64(es3.reshape(N), ed3.reshape(N), snd2, rcv2, h3)
    h4, es4, ed4 = _make_mid(1, 64, 32)(acc3, dr(den3),
                                        b3, g3, o3, W4, asrc4, adst4)
    acc4, den4 = sc32(es4.reshape(N), ed4.reshape(N), snd2, rcv2, h4)
    s = _tc_last(acc4, dr(den4), b4)
    return jnp.concatenate([s.reshape(B, N_NODE * ED), r[:, SLEFT:]], axis=1)


# speedup vs baseline: 54.8072x; 1.1339x over previous
import functools

import jax
import jax.numpy as jnp
from jax import lax
from jax.experimental import pallas as pl
from jax.experimental.pallas import tpu as pltpu
from jax.experimental.pallas import tpu_sc as plsc

N_NODE = 5000
N_FEAT = 128
ED = 32
MAXN = 16
N_EDGE = MAXN * (N_NODE - 1)
SLEFT = N_NODE * N_FEAT
B = 4
N = B * N_NODE
E = B * N_EDGE
NPG = 5056
NPGH = NPG // 2
NPGQ = NPG // 4

G = 128
NROWS = 2560
E_PAD = NROWS * G
PAD = E_PAD - E
N_ACC = 20224
RPT = N_ACC // 16
ZR = 79
GROUPS_PER_TILE = NROWS // 32
IB = 2
NCH = GROUPS_PER_TILE // IB


def _sc_body(C, es_hbm, ed_hbm, snd_hbm, rcv_hbm, h_hbm, acc_out, den_out,
             snd_ch, rcv_ch, ex_v, esg_v, edg_v, rows_v, zbuf, dz,
             es_sh, ed_sh, acc_sh, den_sh, sem, sem2, sem3, sem4, sem5):
    cidx = lax.axis_index("c")
    sidx = lax.axis_index("s")
    row0 = sidx * RPT

    zvec = jnp.zeros((16,), jnp.float32)

    def _zb(i, carry):
        for c in range(C // 16):
            zbuf[i, pl.ds(c * 16, 16)] = zvec
        return carry
    lax.fori_loop(0, ZR, _zb, None)

    def _zd(i, carry):
        dz[pl.ds(i * 16, 16)] = zvec
        return carry
    lax.fori_loop(0, RPT // 16, _zd, None)

    for t in range(RPT // ZR):
        pltpu.sync_copy(zbuf, acc_sh.at[pl.ds(row0 + t * ZR, ZR)])
    pltpu.sync_copy(dz, den_sh.at[pl.ds(row0, RPT)])
    @pl.when(sidx == 0)
    def _stage():
        pltpu.sync_copy(es_hbm, es_sh)
        pltpu.sync_copy(ed_hbm, ed_sh)
    plsc.subcore_barrier()

    base = cidx * (NROWS // 2) + sidx * GROUPS_PER_TILE

    pltpu.async_copy(snd_hbm.at[pl.ds(base, IB)], snd_ch.at[0], sem3)
    pltpu.async_copy(rcv_hbm.at[pl.ds(base, IB)], rcv_ch.at[0], sem3)

    def _chunk(cc, carry):
        q = lax.rem(cc, 2)
        qn = 1 - q
        @pl.when(cc > 0)
        def _drain():
            for b in range(IB):
                pltpu.make_async_copy(ex_v.at[b], den_sh.at[rcv_ch.at[q, b]],
                                      sem4).wait()
                pltpu.make_async_copy(rows_v.at[b], acc_sh.at[rcv_ch.at[q, b]],
                                      sem5).wait()
        pltpu.make_async_copy(snd_hbm.at[pl.ds(base, IB)], snd_ch.at[q],
                              sem3).wait()
        pltpu.make_async_copy(rcv_hbm.at[pl.ds(base, IB)], rcv_ch.at[q],
                              sem3).wait()
        nrow = base + jnp.minimum(cc + 1, NCH - 1) * IB
        pltpu.async_copy(snd_hbm.at[pl.ds(nrow, IB)], snd_ch.at[qn], sem3)
        pltpu.async_copy(rcv_hbm.at[pl.ds(nrow, IB)], rcv_ch.at[qn], sem3)
        cps = []
        for b in range(IB):
            sref = snd_ch.at[q, b]
            rref = rcv_ch.at[q, b]
            cps.append((pltpu.async_copy(h_hbm.at[sref], rows_v.at[b], sem),
                        pltpu.async_copy(es_sh.at[sref], esg_v.at[b], sem2),
                        pltpu.async_copy(ed_sh.at[rref], edg_v.at[b], sem2)))
        for b in range(IB):
            c1, c2, c3 = cps[b]
            c2.wait()
            c3.wait()
            for j in range(G // 16):
                e = esg_v[b, pl.ds(j * 16, 16)] + edg_v[b, pl.ds(j * 16, 16)]
                e = jnp.maximum(e, 0.2 * e)
                ex_v[b, pl.ds(j * 16, 16)] = jnp.exp(e)
            pltpu.async_copy(ex_v.at[b], den_sh.at[rcv_ch.at[q, b]], sem4,
                             add=True)
            c1.wait()

            for j in range(G // 16):
                exv = ex_v[b, pl.ds(j * 16, 16)]
                for kk in range(16):
                    exb = jnp.broadcast_to(exv[kk], (16,))
                    k = j * 16 + kk
                    for c in range(C // 16):
                        rows_v[b, k, pl.ds(c * 16, 16)] = (
                            rows_v[b, k, pl.ds(c * 16, 16)] * exb)
            pltpu.async_copy(rows_v.at[b], acc_sh.at[rcv_ch.at[q, b]], sem5,
                             add=True)
        return carry

    lax.fori_loop(0, NCH, _chunk, None)
    for b in range(IB):
        pltpu.make_async_copy(ex_v.at[b], den_sh.at[rcv_ch.at[0, b]],
                              sem4).wait()
        pltpu.make_async_copy(rows_v.at[b], acc_sh.at[rcv_ch.at[0, b]],
                              sem5).wait()
    pltpu.make_async_copy(snd_hbm.at[pl.ds(base, IB)], snd_ch.at[0], sem3).wait()
    pltpu.make_async_copy(rcv_hbm.at[pl.ds(base, IB)], rcv_ch.at[0], sem3).wait()
    plsc.subcore_barrier()

    pltpu.sync_copy(acc_sh.at[pl.ds(row0, RPT)],
                    acc_out.at[cidx, pl.ds(row0, RPT)])
    pltpu.sync_copy(den_sh.at[pl.ds(row0, RPT)],
                    den_out.at[pl.ds(cidx * N_ACC + row0, RPT)])


@functools.lru_cache(maxsize=None)
def _make_sc(C):
    mesh = plsc.VectorSubcoreMesh(core_axis_name="c", subcore_axis_name="s")
    return pl.kernel(
        functools.partial(_sc_body, C),
        out_type=(jax.ShapeDtypeStruct((2, N_ACC, C), jnp.float32),
                  jax.ShapeDtypeStruct((2 * N_ACC,), jnp.float32)),
        mesh=mesh,
        scratch_types=[
            pltpu.VMEM((2, IB, G), jnp.int32),
            pltpu.VMEM((2, IB, G), jnp.int32),
            pltpu.VMEM((IB, G), jnp.float32),
            pltpu.VMEM((IB, G), jnp.float32),
            pltpu.VMEM((IB, G), jnp.float32),
            pltpu.VMEM((IB, G, C), jnp.float32),
            pltpu.VMEM((ZR, C), jnp.float32),
            pltpu.VMEM((RPT,), jnp.float32),
            pltpu.VMEM_SHARED((N_ACC,), jnp.float32),
            pltpu.VMEM_SHARED((N_ACC,), jnp.float32),
            pltpu.VMEM_SHARED((N_ACC, C), jnp.float32),
            pltpu.VMEM_SHARED((N_ACC,), jnp.float32),
            pltpu.SemaphoreType.DMA,
            pltpu.SemaphoreType.DMA,
            pltpu.SemaphoreType.DMA,
            pltpu.SemaphoreType.DMA,
            pltpu.SemaphoreType.DMA,
        ],
        compiler_params=pltpu.CompilerParams(needs_layout_passes=False,
                                             use_tc_tiling_on_sc=False),
        name=f"gat_edge_sweep_c{C}",
    )


def _vec_spec(n):
    return pl.BlockSpec((n,), lambda i: (0,))


def _mat_spec(a, b_):
    return pl.BlockSpec((a, b_), lambda i: (0, 0))


HREAL = N_NODE // 2


def _bd(w, ci, co, p):
    wt = jnp.concatenate([jnp.concatenate([w] * p, 1)] * p, 0)
    row = lax.broadcasted_iota(jnp.int32, (p * ci, p * co), 0)
    col = lax.broadcasted_iota(jnp.int32, (p * ci, p * co), 1)
    return jnp.where(row // ci == col // co, wt, 0.0)


def _smat(wa, ci, p):
    wt = jnp.concatenate([wa] * p)[:, None]
    row = lax.broadcasted_iota(jnp.int32, (p * ci, p), 0)
    col = lax.broadcasted_iota(jnp.int32, (p * ci, p), 1)
    return jnp.where(row // ci == col, jnp.broadcast_to(wt, (p * ci, p)), 0.0)


def _dmat(q, w):
    row = lax.broadcasted_iota(jnp.int32, (q, q * w), 0)
    col = lax.broadcasted_iota(jnp.int32, (q, q * w), 1)
    return jnp.where(col // w == row, 1.0, 0.0)


def _pad_store(ref, val, width):
    ref[0, :HREAL] = val
    ref[0, HREAL:] = jnp.zeros((NPGH - HREAL, width), jnp.float32)


def _first_body(x_ref, w_ref, asrc_ref, adst_ref, ha_ref, hb_ref, es_ref, ed_ref):
    x = x_ref[0]
    w = w_ref[...]
    ha = jnp.dot(x, _bd(w[:, :64], N_FEAT, 64, 2),
                 preferred_element_type=jnp.float32)
    hb = jnp.dot(x, _bd(w[:, 64:], N_FEAT, 64, 2),
                 preferred_element_type=jnp.float32)
    wa = jnp.dot(w, asrc_ref[...], preferred_element_type=jnp.float32)
    wd = jnp.dot(w, adst_ref[...], preferred_element_type=jnp.float32)
    _pad_store(ha_ref, ha, 128)
    _pad_store(hb_ref, hb, 128)
    _pad_store(es_ref, jnp.dot(x, _smat(wa, N_FEAT, 2),
                               preferred_element_type=jnp.float32), 2)
    _pad_store(ed_ref, jnp.dot(x, _smat(wd, N_FEAT, 2),
                               preferred_element_type=jnp.float32), 2)


_tc_first = pl.pallas_call(
    _first_body,
    grid=(B,),
    in_specs=[pl.BlockSpec((1, HREAL, 2 * N_FEAT), lambda i: (i, 0, 0)),
              _mat_spec(N_FEAT, N_FEAT), _vec_spec(N_FEAT), _vec_spec(N_FEAT)],
    out_specs=(pl.BlockSpec((1, NPGH, 128), lambda i: (i, 0, 0)),
               pl.BlockSpec((1, NPGH, 128), lambda i: (i, 0, 0)),
               pl.BlockSpec((1, NPGH, 2), lambda i: (i, 0, 0)),
               pl.BlockSpec((1, NPGH, 2), lambda i: (i, 0, 0))),
    out_shape=(jax.ShapeDtypeStruct((B, NPGH, 128), jnp.float32),
               jax.ShapeDtypeStruct((B, NPGH, 128), jnp.float32),
               jax.ShapeDtypeStruct((B, NPGH, 2), jnp.float32),
               jax.ShapeDtypeStruct((B, NPGH, 2), jnp.float32)),
    name="gat_first",
)


def _mid_body(nacc, cin, cout, refs):
    accs = refs[:nacc]
    (den_ref, b_ref, g_ref, o_ref, w_ref, asrc_ref, adst_ref,
     h_ref, es_ref, ed_ref) = refs[nacc:]
    if nacc == 1:
        s = accs[0][0, 0] + accs[0][1, 0]
    else:
        sa = accs[0][0, 0] + accs[0][1, 0]
        sb = accs[1][0, 0] + accs[1][1, 0]
        cat = jnp.concatenate([sa, sb], axis=1)
        col = lax.broadcasted_iota(jnp.int32, (256, 256), 1)
        src = jnp.where(col < 64, col,
                        jnp.where(col < 128, col + 64,
                                  jnp.where(col < 192, col - 64, col)))
        row = lax.broadcasted_iota(jnp.int32, (256, 256), 0)
        perm = jnp.where(row == src, 1.0, 0.0)
        s = jnp.dot(cat, perm, preferred_element_type=jnp.float32)
    w2 = 2 * cin
    d2 = den_ref[0, 0] + den_ref[1, 0]
    d = jnp.dot(d2, _dmat(2, cin), preferred_element_type=jnp.float32)
    x = s[:HREAL] / (d[:HREAL] + 1e-16)
    x = x + jnp.concatenate([b_ref[...]] * 2)
    mu_l = jnp.mean(x, axis=0)
    mu_c = (mu_l[:cin] + mu_l[cin:]) * 0.5
    mu = jnp.concatenate([mu_c, mu_c])
    xc = x - mu
    v_l = jnp.mean(xc * xc, axis=0)
    v_c = (v_l[:cin] + v_l[cin:]) * 0.5
    var = jnp.concatenate([v_c, v_c])
    x = xc * lax.rsqrt(var + 1e-5)
    x = x * jnp.concatenate([g_ref[...]] * 2) + jnp.concatenate([o_ref[...]] * 2)
    x = jnp.maximum(x, 0.0)
    w = w_ref[...]
    h = jnp.dot(x, _bd(w, cin, cout, 2), preferred_element_type=jnp.float32)
    _pad_store(h_ref, h, 2 * cout)
    wa = jnp.dot(w, asrc_ref[...], preferred_element_type=jnp.float32)
    wd = jnp.dot(w, adst_ref[...], preferred_element_type=jnp.float32)
    _pad_store(es_ref, jnp.dot(x, _smat(wa, cin, 2),
                               preferred_element_type=jnp.float32), 2)
    _pad_store(ed_ref, jnp.dot(x, _smat(wd, cin, 2),
                               preferred_element_type=jnp.float32), 2)


_ACC_SPEC = pl.BlockSpec((2, 1, NPGH, 128), lambda i: (0, i, 0, 0))
_DEN_SPEC = pl.BlockSpec((2, 1, NPGH, 2), lambda i: (0, i, 0, 0))


@functools.lru_cache(maxsize=None)
def _make_mid(nacc, cin, cout):
    def body(*refs):
        _mid_body(nacc, cin, cout, refs)
    return pl.pallas_call(
        body,
        grid=(B,),
        in_specs=[_ACC_SPEC] * nacc + [
            _DEN_SPEC, _vec_spec(cin), _vec_spec(cin), _vec_spec(cin),
            _mat_spec(cin, cout), _vec_spec(cout), _vec_spec(cout)],
        out_specs=(pl.BlockSpec((1, NPGH, 2 * cout), lambda i: (i, 0, 0)),
                   pl.BlockSpec((1, NPGH, 2), lambda i: (i, 0, 0)),
                   pl.BlockSpec((1, NPGH, 2), lambda i: (i, 0, 0))),
        out_shape=(jax.ShapeDtypeStruct((B, NPGH, 2 * cout), jnp.float32),
                   jax.ShapeDtypeStruct((B, NPGH, 2), jnp.float32),
                   jax.ShapeDtypeStruct((B, NPGH, 2), jnp.float32)),
        name=f"gat_mid_{cout}",
    )


def _last_body(acc_ref, den_ref, b_ref, s_ref):
    s = acc_ref[0, 0] + acc_ref[1, 0]
    d4 = den_ref[0, 0] + den_ref[1, 0]
    d = jnp.dot(d4, _dmat(4, ED), preferred_element_type=jnp.float32)
    out = s / (d + 1e-16) + jnp.concatenate([b_ref[...]] * 4)
    s_ref[0] = out[:N_NODE // 4]


_tc_last = pl.pallas_call(
    _last_body,
    grid=(B,),
    in_specs=[pl.BlockSpec((2, 1, NPGQ, 128), lambda i: (0, i, 0, 0)),
              pl.BlockSpec((2, 1, NPGQ, 4), lambda i: (0, i, 0, 0)),
              _vec_spec(ED)],
    out_specs=pl.BlockSpec((1, N_NODE // 4, 128), lambda i: (i, 0, 0)),
    out_shape=jax.ShapeDtypeStruct((B, N_NODE // 4, 128), jnp.float32),
    name="gat_last",
)


def kernel(r, W1, asrc1, adst1, b1, g1, o1, W2, asrc2, adst2, b2, g2, o2,
           W3, asrc3, adst3, b3, g3, o3, W4, asrc4, adst4, b4):
    nodes = r[:, :SLEFT].reshape(B, HREAL, 2 * N_FEAT)
    off = (jnp.arange(B, dtype=jnp.int32) * NPG)[:, None]
    snd = (r[:, SLEFT:SLEFT + N_EDGE].astype(jnp.int32) + off).reshape(-1)
    rcv = (r[:, SLEFT + N_EDGE:SLEFT + 2 * N_EDGE].astype(jnp.int32) + off).reshape(-1)
    ar = jnp.arange(PAD, dtype=jnp.int32)
    pad_g = ar % B
    pad_s = (ar * 97) % N_NODE + pad_g * NPG
    pad_r = N_NODE + (ar // B) % (NPG - N_NODE) + pad_g * NPG
    snd2 = jnp.concatenate([snd, pad_s]).reshape(NROWS, G)
    rcv2 = jnp.concatenate([rcv, pad_r]).reshape(NROWS, G)

    sc64 = _make_sc(64)
    sc32 = _make_sc(32)

    def denp(den, q=2):
        return den.reshape(2, B, NPG // q, q)

    ha, hb, es, ed = _tc_first(nodes, W1, asrc1, adst1)
    es, ed = es.reshape(N_ACC), ed.reshape(N_ACC)
    acc_a, den1 = sc64(es, ed, snd2, rcv2, ha.reshape(N_ACC, 64))
    acc_b, _ = sc64(es, ed, snd2, rcv2, hb.reshape(N_ACC, 64))
    h2, es2, ed2 = _make_mid(2, 128, 64)(
        acc_a.reshape(2, B, NPGH, 128), acc_b.reshape(2, B, NPGH, 128),
        denp(den1), b1, g1, o1, W2, asrc2, adst2)
    acc2, den2 = sc64(es2.reshape(N_ACC), ed2.reshape(N_ACC), snd2, rcv2,
                      h2.reshape(N_ACC, 64))
    h3, es3, ed3 = _make_mid(1, 64, 64)(
        acc2.reshape(2, B, NPGH, 128), denp(den2), b2, g2, o2, W3, asrc3, adst3)
    acc3, den3 = sc64(es3.reshape(N_ACC), ed3.reshape(N_ACC), snd2, rcv2,
                      h3.reshape(N_ACC, 64))
    h4, es4, ed4 = _make_mid(1, 64, 32)(
        acc3.reshape(2, B, NPGH, 128), denp(den3), b3, g3, o3, W4, asrc4, adst4)
    acc4, den4 = sc32(es4.reshape(N_ACC), ed4.reshape(N_ACC), snd2, rcv2,
                      h4.reshape(N_ACC, 32))
    s = _tc_last(acc4.reshape(2, B, NPGQ, 128), denp(den4, 4), b4)
    return jnp.concatenate([s.reshape(B, N_NODE * ED), r[:, SLEFT:]], axis=1)
